# baseline (speedup 1.0000x reference)
import functools

import jax
import jax.numpy as jnp
from jax import lax
from jax.experimental import pallas as pl
from jax.experimental.pallas import tpu as pltpu
from jax.experimental.pallas import tpu_sc as plsc

NUM_EXPERTS = 16
TOP_K = 2
HIDDEN = 1024
MOE_FF = 768
TOKENS = 2048

NUM_ASSIGN = TOKENS * TOP_K
BLK = 128
NB = NUM_ASSIGN // BLK + NUM_EXPERTS
NR = NB * BLK

_SC_CORES = 2
_SC_SUBCORES = 16
_NW = _SC_CORES * _SC_SUBCORES

_DISPATCH_ROWS = NR // _NW
_DISPATCH_CHUNK = 64
_COMBINE_TOKENS = TOKENS // _NW
_COMBINE_CHUNK = 32


def _router_body(x_ref, gate_ref, wsel_ref, onehot_ref):
    logits = jnp.dot(x_ref[...], gate_ref[...], preferred_element_type=jnp.float32)
    probs = jax.nn.softmax(logits, axis=-1)
    lane = lax.broadcasted_iota(jnp.int32, probs.shape, 1)
    m1 = jnp.max(probs, axis=-1, keepdims=True)
    i1 = jnp.min(jnp.where(probs == m1, lane, NUM_EXPERTS), axis=-1, keepdims=True)
    masked = jnp.where(lane == i1, -jnp.inf, probs)
    m2 = jnp.max(masked, axis=-1, keepdims=True)
    i2 = jnp.min(jnp.where(masked == m2, lane, NUM_EXPERTS), axis=-1, keepdims=True)
    denom = m1 + m2
    sel = (lane == i1) | (lane == i2)
    w = jnp.where(lane == i1, m1, jnp.where(lane == i2, m2, 0.0)) / denom
    wsel_ref[...] = w
    onehot_ref[...] = sel.astype(jnp.float32)


def _router(x, gate_w):
    return pl.pallas_call(
        _router_body,
        out_shape=(
            jax.ShapeDtypeStruct((TOKENS, NUM_EXPERTS), jnp.float32),
            jax.ShapeDtypeStruct((TOKENS, NUM_EXPERTS), jnp.float32),
        ),
    )(x, gate_w)


def _route_metadata(wsel, onehot):
    mask = onehot > 0.5
    maski = mask.astype(jnp.int32)
    csum = jnp.cumsum(maski, axis=0)
    counts = csum[-1]
    blocks = (counts + BLK - 1) // BLK
    blockstart = jnp.concatenate([jnp.zeros((1,), jnp.int32),
                                  jnp.cumsum(blocks)[:-1]]).astype(jnp.int32)
    base = blockstart * BLK

    posmat = base[None, :] + csum - 1
    flatpos = jnp.where(mask, posmat, NR)
    tok = lax.broadcasted_iota(jnp.int32, mask.shape, 0)
    src = jnp.zeros((NR + 1,), jnp.int32).at[flatpos.reshape(-1)].set(
        tok.reshape(-1), mode="drop")
    src_token = src[:NR]

    marks = jnp.zeros((NB,), jnp.int32).at[blockstart].max(
        jnp.arange(NUM_EXPERTS, dtype=jnp.int32), mode="drop")
    block_expert = lax.cummax(marks)
    bidx = jnp.arange(NB, dtype=jnp.int32)
    block_valid = jnp.clip(
        counts[block_expert] - (bidx - blockstart[block_expert]) * BLK, 0, BLK)

    first = jnp.argmax(maski, axis=1)
    last = (NUM_EXPERTS - 1) - jnp.argmax(maski[:, ::-1], axis=1)
    tidx = jnp.arange(TOKENS)
    pos0 = posmat[tidx, first]
    pos1 = posmat[tidx, last]
    w0 = wsel[tidx, first]
    w1 = wsel[tidx, last]
    return (src_token, block_expert, block_valid,
            pos0.astype(jnp.int32), pos1.astype(jnp.int32), w0, w1)


def _sc_mesh():
    return plsc.VectorSubcoreMesh(core_axis_name="c", subcore_axis_name="s")


@functools.cache
def _make_sc_dispatch():
    @functools.partial(
        pl.kernel,
        mesh=_sc_mesh(),
        out_type=jax.ShapeDtypeStruct((NR, HIDDEN), jnp.float32),
        scratch_types=[
            pltpu.VMEM((_DISPATCH_ROWS,), jnp.int32),
            pltpu.VMEM((_DISPATCH_CHUNK, HIDDEN), jnp.float32),
            pltpu.SemaphoreType.DMA,
        ],
    )
    def _sc_dispatch(hid_hbm, src_hbm, out_hbm, idx_v, rows_v, sem):
        wid = lax.axis_index("s") * _SC_CORES + lax.axis_index("c")
        row_base = wid * _DISPATCH_ROWS
        pltpu.sync_copy(src_hbm.at[pl.ds(row_base, _DISPATCH_ROWS)], idx_v)

        def step(c, _):
            pltpu.async_copy(
                hid_hbm.at[idx_v.at[pl.ds(c * _DISPATCH_CHUNK, _DISPATCH_CHUNK)]],
                rows_v, sem).wait()
            pltpu.sync_copy(
                rows_v,
                out_hbm.at[pl.ds(row_base + c * _DISPATCH_CHUNK, _DISPATCH_CHUNK)])
            return 0

        lax.fori_loop(0, _DISPATCH_ROWS // _DISPATCH_CHUNK, step, 0)

    return _sc_dispatch


def _group_ffn_body(be_ref, valid_ref, x_ref, wg_ref, wu_ref, wd_ref, out_ref):
    b = pl.program_id(0)

    @pl.when(valid_ref[b] > 0)
    def _():
        x = x_ref[...]
        g = jnp.dot(x, wg_ref[0], preferred_element_type=jnp.float32)
        u = jnp.dot(x, wu_ref[0], preferred_element_type=jnp.float32)
        h = (g * jax.nn.sigmoid(g)) * u
        out_ref[...] = jnp.dot(h, wd_ref[0], preferred_element_type=jnp.float32)


def _group_ffn(x_pad, w_gate, w_up, w_down, block_expert, block_valid):
    grid_spec = pltpu.PrefetchScalarGridSpec(
        num_scalar_prefetch=2,
        grid=(NB,),
        in_specs=[
            pl.BlockSpec((BLK, HIDDEN), lambda b, be, vd: (b, 0)),
            pl.BlockSpec((1, HIDDEN, MOE_FF), lambda b, be, vd: (be[b], 0, 0)),
            pl.BlockSpec((1, HIDDEN, MOE_FF), lambda b, be, vd: (be[b], 0, 0)),
            pl.BlockSpec((1, MOE_FF, HIDDEN), lambda b, be, vd: (be[b], 0, 0)),
        ],
        out_specs=pl.BlockSpec((BLK, HIDDEN), lambda b, be, vd: (b, 0)),
    )
    return pl.pallas_call(
        _group_ffn_body,
        grid_spec=grid_spec,
        out_shape=jax.ShapeDtypeStruct((NR, HIDDEN), jnp.float32),
    )(block_expert, block_valid, x_pad, w_gate, w_up, w_down)


@functools.cache
def _make_sc_combine():
    @functools.partial(
        pl.kernel,
        mesh=_sc_mesh(),
        out_type=jax.ShapeDtypeStruct((TOKENS, HIDDEN), jnp.float32),
        scratch_types=[
            pltpu.VMEM((_COMBINE_TOKENS,), jnp.int32),
            pltpu.VMEM((_COMBINE_TOKENS,), jnp.int32),
            pltpu.VMEM((_COMBINE_TOKENS, 16), jnp.float32),
            pltpu.VMEM((_COMBINE_TOKENS, 16), jnp.float32),
            pltpu.VMEM((_COMBINE_CHUNK, HIDDEN), jnp.float32),
            pltpu.VMEM((_COMBINE_CHUNK, HIDDEN), jnp.float32),
            pltpu.VMEM((_COMBINE_CHUNK, HIDDEN), jnp.float32),
            pltpu.SemaphoreType.DMA,
            pltpu.SemaphoreType.DMA,
        ],
    )
    def _sc_combine(y_hbm, pos0_hbm, pos1_hbm, w0_hbm, w1_hbm, out_hbm,
                    pos0_v, pos1_v, w0_v, w1_v, y0_v, y1_v, o_v, sem0, sem1):
        wid = lax.axis_index("s") * _SC_CORES + lax.axis_index("c")
        tok_base = wid * _COMBINE_TOKENS
        pltpu.sync_copy(pos0_hbm.at[pl.ds(tok_base, _COMBINE_TOKENS)], pos0_v)
        pltpu.sync_copy(pos1_hbm.at[pl.ds(tok_base, _COMBINE_TOKENS)], pos1_v)
        pltpu.sync_copy(w0_hbm.at[pl.ds(tok_base, _COMBINE_TOKENS)], w0_v)
        pltpu.sync_copy(w1_hbm.at[pl.ds(tok_base, _COMBINE_TOKENS)], w1_v)

        def chunk(c, _):
            off = c * _COMBINE_CHUNK
            cp0 = pltpu.async_copy(
                y_hbm.at[pos0_v.at[pl.ds(off, _COMBINE_CHUNK)]], y0_v, sem0)
            cp1 = pltpu.async_copy(
                y_hbm.at[pos1_v.at[pl.ds(off, _COMBINE_CHUNK)]], y1_v, sem1)
            cp0.wait()
            cp1.wait()

            def token(j, _):
                wa = w0_v[off + j, :]
                wb = w1_v[off + j, :]
                for i in range(HIDDEN // 16):
                    sl = pl.ds(i * 16, 16)
                    o_v[j, sl] = wa * y0_v[j, sl] + wb * y1_v[j, sl]
                return 0

            lax.fori_loop(0, _COMBINE_CHUNK, token, 0)
            pltpu.sync_copy(
                o_v, out_hbm.at[pl.ds(tok_base + off, _COMBINE_CHUNK)])
            return 0

        lax.fori_loop(0, _COMBINE_TOKENS // _COMBINE_CHUNK, chunk, 0)

    return _sc_combine


@jax.jit
def kernel(hidden_states, gate_w, w_gate, w_up, w_down):
    wsel, onehot = _router(hidden_states, gate_w)
    (src_token, block_expert, block_valid,
     pos0, pos1, w0, w1) = _route_metadata(wsel, onehot)
    x_pad = _make_sc_dispatch()(hidden_states, src_token)
    y_pad = _group_ffn(x_pad, w_gate, w_up, w_down, block_expert, block_valid)
    w0mat = jnp.tile(w0[:, None], (1, 16))
    w1mat = jnp.tile(w1[:, None], (1, 16))
    return _make_sc_combine()(y_pad, pos0, pos1, w0mat, w1mat)
